# Initial kernel scaffold; baseline (speedup 1.0000x reference)
import functools

import jax
import jax.numpy as jnp
from jax import lax
from jax.experimental import pallas as pl
from jax.experimental.pallas import tpu as pltpu
from jax.experimental.pallas import tpu_sc as plsc

EMBED = 64
G = 128
NC, NS = 2, 16
NW = NC * NS


def _body(idx_hbm, w_hbm, out_hbm, idx_v, rows_v, in_sem, chunks):
    wid = lax.axis_index("s") * NC + lax.axis_index("c")
    base = wid * chunks
    pltpu.sync_copy(idx_hbm.at[pl.ds(base, chunks)], idx_v)

    zeros16 = jnp.zeros((16,), jnp.float32)
    lane = lax.iota(jnp.int32, 16)

    def step(j, carry):
        pltpu.async_copy(w_hbm.at[idx_v.at[j]], rows_v, in_sem).wait()
        m = idx_v[j, pl.ds(0, 16)]
        for k in range(1, G // 16):
            m = jnp.minimum(m, idx_v[j, pl.ds(k * 16, 16)])
        gmin = jnp.min(m)

        @pl.when(gmin <= 0)
        def _fixup():
            for sg in range(G // 16):
                iv = idx_v[j, pl.ds(sg * 16, 16)]
                msk = iv <= 0
                rows = lane + sg * 16

                def zero_col(c, c2):
                    plsc.store_scatter(
                        rows_v,
                        [rows, jnp.full((16,), 0, jnp.int32) + c],
                        zeros16,
                        mask=msk,
                    )
                    return c2

                lax.fori_loop(0, EMBED, zero_col, 0)

        pltpu.sync_copy(rows_v, out_hbm.at[pl.ds((base + j) * G, G)])
        return carry

    lax.fori_loop(0, chunks, step, 0)


@functools.partial(jax.jit, static_argnames=("chunks",))
def _embed_call(idx2, w, chunks):
    total = chunks * NW * G
    mesh = plsc.VectorSubcoreMesh(core_axis_name="c", subcore_axis_name="s")
    return pl.kernel(
        functools.partial(_body, chunks=chunks),
        out_type=jax.ShapeDtypeStruct((total, EMBED), jnp.float32),
        mesh=mesh,
        scratch_types=[
            pltpu.VMEM((chunks, G), jnp.int32),
            pltpu.VMEM((G, EMBED), jnp.float32),
            pltpu.SemaphoreType.DMA,
        ],
    )(idx2, w)


def kernel(x, W):
    b, h, _ = x.shape
    total = b * h
    assert total % (NW * G) == 0
    chunks = total // (NW * G)
    idx2 = x.reshape(total // G, G).astype(jnp.int32)
    out = _embed_call(idx2, W, chunks)
    return out.reshape(b, h, EMBED)


# speedup vs baseline: 1.6747x; 1.6747x over previous
import functools

import jax
import jax.numpy as jnp
from jax import lax
from jax.experimental import pallas as pl
from jax.experimental.pallas import tpu as pltpu
from jax.experimental.pallas import tpu_sc as plsc

EMBED = 64
G = 128
NC, NS = 2, 16
NW = NC * NS


def _body(idx_hbm, w_hbm, out_hbm, idx_v, rows_v, in_sem, chunks):
    wid = lax.axis_index("s") * NC + lax.axis_index("c")
    base = wid * chunks
    pltpu.sync_copy(idx_hbm.at[pl.ds(base, chunks)], idx_v)

    zeros16 = jnp.zeros((16,), jnp.float32)
    lane = lax.iota(jnp.int32, 16)

    def step(j, carry):
        pltpu.async_copy(w_hbm.at[idx_v.at[j]], rows_v, in_sem).wait()
        m = idx_v[j, pl.ds(0, 16)]
        for k in range(1, G // 16):
            m = jnp.minimum(m, idx_v[j, pl.ds(k * 16, 16)])
        gmin = m[0]
        for i in range(1, 16):
            gmin = jnp.minimum(gmin, m[i])

        @pl.when(gmin <= 0)
        def _fixup():
            for sg in range(G // 16):
                iv = idx_v[j, pl.ds(sg * 16, 16)]
                for i in range(16):

                    @pl.when(iv[i] <= 0)
                    def _zero_row(sg=sg, i=i):
                        for q in range(EMBED // 16):
                            rows_v[sg * 16 + i, pl.ds(q * 16, 16)] = zeros16

        pltpu.sync_copy(rows_v, out_hbm.at[pl.ds((base + j) * G, G)])
        return carry

    lax.fori_loop(0, chunks, step, 0)


@functools.partial(jax.jit, static_argnames=("chunks",))
def _embed_call(idx2, w, chunks):
    total = chunks * NW * G
    mesh = plsc.VectorSubcoreMesh(core_axis_name="c", subcore_axis_name="s")
    return pl.kernel(
        functools.partial(_body, chunks=chunks),
        out_type=jax.ShapeDtypeStruct((total, EMBED), jnp.float32),
        mesh=mesh,
        scratch_types=[
            pltpu.VMEM((chunks, G), jnp.int32),
            pltpu.VMEM((G, EMBED), jnp.float32),
            pltpu.SemaphoreType.DMA,
        ],
        compiler_params=pltpu.CompilerParams(use_tc_tiling_on_sc=False),
    )(idx2, w)


def kernel(x, W):
    b, h, _ = x.shape
    total = b * h
    assert total % (NW * G) == 0
    chunks = total // (NW * G)
    idx2 = x.reshape(total // G, G).astype(jnp.int32)
    out = _embed_call(idx2, W, chunks)
    return out.reshape(b, h, EMBED)


# speedup vs baseline: 1.7413x; 1.0398x over previous
import functools

import jax
import jax.numpy as jnp
from jax import lax
from jax.experimental import pallas as pl
from jax.experimental.pallas import tpu as pltpu
from jax.experimental.pallas import tpu_sc as plsc

EMBED = 64
HIST = 50
G = 128
NC, NS = 2, 16
NW = NC * NS


def _body(xt_hbm, w_hbm, out_hbm, xv, rows_v, in_sem, bpw):
    batch = bpw * NW
    wid = lax.axis_index("s") * NC + lax.axis_index("c")
    b0 = wid * bpw
    pltpu.sync_copy(xt_hbm.at[:, pl.ds(b0, bpw)], xv)

    zeros16 = jnp.zeros((16,), jnp.float32)

    def step(t, carry):
        h = t // (bpw // G)
        k = t % (bpw // G)
        kb = k * G
        idx_ref = xv.at[h, pl.ds(kb, G)]
        pltpu.async_copy(w_hbm.at[idx_ref], rows_v, in_sem).wait()
        m = xv[h, pl.ds(kb, 16)]
        for q in range(1, G // 16):
            m = jnp.minimum(m, xv[h, pl.ds(kb + q * 16, 16)])
        gmin = m[0]
        for i in range(1, 16):
            gmin = jnp.minimum(gmin, m[i])

        @pl.when(gmin <= 0)
        def _fixup():
            for sg in range(G // 16):
                iv = xv[h, pl.ds(kb + sg * 16, 16)]
                for i in range(16):

                    @pl.when(iv[i] <= 0)
                    def _zero_row(sg=sg, i=i):
                        for q in range(EMBED // 16):
                            rows_v[sg * 16 + i, pl.ds(q * 16, 16)] = zeros16

        pltpu.sync_copy(rows_v, out_hbm.at[pl.ds(h * batch + b0 + kb, G)])
        return carry

    lax.fori_loop(0, HIST * (bpw // G), step, 0)


@functools.partial(jax.jit, static_argnames=("bpw",))
def _embed_call(xt, w, bpw):
    batch = bpw * NW
    mesh = plsc.VectorSubcoreMesh(core_axis_name="c", subcore_axis_name="s")
    return pl.kernel(
        functools.partial(_body, bpw=bpw),
        out_type=jax.ShapeDtypeStruct((HIST * batch, EMBED), jnp.float32),
        mesh=mesh,
        scratch_types=[
            pltpu.VMEM((HIST, bpw), jnp.int32),
            pltpu.VMEM((G, EMBED), jnp.float32),
            pltpu.SemaphoreType.DMA,
        ],
        compiler_params=pltpu.CompilerParams(use_tc_tiling_on_sc=False),
    )(xt, w)


def kernel(x, W):
    b, h, _ = x.shape
    assert h == HIST and b % (NW * G) == 0
    bpw = b // NW
    xt = jnp.transpose(x, (1, 0, 2))[:, :, 0].astype(jnp.int32)
    out = _embed_call(xt, W, bpw)
    return jnp.transpose(out.reshape(HIST, b, EMBED), (1, 0, 2))


# speedup vs baseline: 1.8294x; 1.0506x over previous
import functools

import jax
import jax.numpy as jnp
from jax import lax
from jax.experimental import pallas as pl
from jax.experimental.pallas import tpu as pltpu
from jax.experimental.pallas import tpu_sc as plsc

EMBED = 64
HIST = 50
G = 128
NC, NS = 2, 16
NW = NC * NS


def _body(xt_hbm, w_hbm, out_hbm, xv, rows_v, in_sem, bpw):
    batch = bpw * NW
    wid = lax.axis_index("s") * NC + lax.axis_index("c")
    b0 = wid * bpw
    pltpu.sync_copy(xt_hbm.at[:, pl.ds(b0, bpw)], xv)

    zeros16 = jnp.zeros((16,), jnp.float32)

    def step(t, carry):
        h = t // (bpw // G)
        k = t % (bpw // G)
        kb = k * G
        m = xv[h, pl.ds(kb, 16)]
        xv[h, pl.ds(kb, 16)] = m + m
        for q in range(1, G // 16):
            v = xv[h, pl.ds(kb + q * 16, 16)]
            xv[h, pl.ds(kb + q * 16, 16)] = v + v
            m = jnp.minimum(m, v)
        idx_ref = xv.at[h, pl.ds(kb, G)]
        pltpu.async_copy(w_hbm.at[idx_ref], rows_v, in_sem).wait()
        gmin = m[0]
        for i in range(1, 16):
            gmin = jnp.minimum(gmin, m[i])

        @pl.when(gmin <= 0)
        def _fixup():
            for sg in range(G // 16):
                iv = xv[h, pl.ds(kb + sg * 16, 16)]
                for i in range(16):

                    @pl.when(iv[i] <= 0)
                    def _zero_row(sg=sg, i=i):
                        for q in range(EMBED // 16):
                            rows_v[sg * 16 + i, pl.ds(q * 16, 16)] = zeros16

        pltpu.sync_copy(rows_v, out_hbm.at[pl.ds(h * batch + b0 + kb, G)])
        return carry

    lax.fori_loop(0, HIST * (bpw // G), step, 0)


@functools.partial(jax.jit, static_argnames=("bpw",))
def _embed_call(xt, w, bpw):
    batch = bpw * NW
    mesh = plsc.VectorSubcoreMesh(core_axis_name="c", subcore_axis_name="s")
    return pl.kernel(
        functools.partial(_body, bpw=bpw),
        out_type=jax.ShapeDtypeStruct((HIST * batch, EMBED), jnp.float32),
        mesh=mesh,
        scratch_types=[
            pltpu.VMEM((HIST, bpw), jnp.int32),
            pltpu.VMEM((G, EMBED), jnp.float32),
            pltpu.SemaphoreType.DMA,
        ],
        compiler_params=pltpu.CompilerParams(use_tc_tiling_on_sc=False),
    )(xt, w)


def kernel(x, W):
    b, h, _ = x.shape
    assert h == HIST and b % (NW * G) == 0
    bpw = b // NW
    xt = jnp.transpose(x, (1, 0, 2))[:, :, 0].astype(jnp.int32)
    w2 = jnp.pad(W, ((0, 0), (0, EMBED))).reshape(2 * W.shape[0], EMBED)
    out = _embed_call(xt, w2, bpw)
    return jnp.transpose(out.reshape(HIST, b, EMBED), (1, 0, 2))


# speedup vs baseline: 2.0393x; 1.1148x over previous
import functools

import jax
import jax.numpy as jnp
from jax import lax
from jax.experimental import pallas as pl
from jax.experimental.pallas import tpu as pltpu
from jax.experimental.pallas import tpu_sc as plsc

EMBED = 64
HIST = 50
G = 128
NC, NS = 2, 16
NW = NC * NS


NBUF = 4


def _body(xt_hbm, w_hbm, out_hbm, xv, rows_v, in_sems, out_sems, bpw):
    batch = bpw * NW
    kph = bpw // G
    nsteps = HIST * kph
    wid = lax.axis_index("s") * NC + lax.axis_index("c")
    b0 = wid * bpw
    pltpu.sync_copy(xt_hbm.at[:, pl.ds(b0, bpw)], xv)

    zeros16 = jnp.zeros((16,), jnp.float32)

    def dbl(i, carry):
        h = i // (bpw // 16)
        kb = (i % (bpw // 16)) * 16
        v = xv[h, pl.ds(kb, 16)]
        xv[h, pl.ds(kb, 16)] = v + v
        return carry

    lax.fori_loop(0, HIST * (bpw // 16), dbl, 0)

    def issue_gather(t):
        h = t // kph
        kb = (t % kph) * G
        pltpu.async_copy(
            w_hbm.at[xv.at[h, pl.ds(kb, G)]],
            rows_v.at[t % NBUF],
            in_sems.at[t % NBUF],
        )

    def step(t, carry):
        h = t // kph
        kb = (t % kph) * G
        slot = t % NBUF
        pltpu.make_async_copy(
            w_hbm.at[xv.at[h, pl.ds(kb, G)]], rows_v.at[slot], in_sems.at[slot]
        ).wait()
        m = xv[h, pl.ds(kb, 16)]
        for q in range(1, G // 16):
            m = jnp.minimum(m, xv[h, pl.ds(kb + q * 16, 16)])
        gmin = m[0]
        for i in range(1, 16):
            gmin = jnp.minimum(gmin, m[i])

        @pl.when(gmin <= 0)
        def _fixup():
            for sg in range(G // 16):
                iv = xv[h, pl.ds(kb + sg * 16, 16)]
                for i in range(16):

                    @pl.when(iv[i] <= 0)
                    def _zero_row(sg=sg, i=i):
                        for q in range(EMBED // 16):
                            rows_v[slot, sg * 16 + i, pl.ds(q * 16, 16)] = zeros16

        dst = out_hbm.at[pl.ds(h * batch + b0 + kb, G)]
        pltpu.async_copy(rows_v.at[slot], dst, out_sems.at[slot])

        @pl.when(t + 2 < nsteps)
        def _ahead():
            @pl.when(t >= 2)
            def _drain():
                t2 = t - 2
                h2 = t2 // kph
                kb2 = (t2 % kph) * G
                dst2 = out_hbm.at[pl.ds(h2 * batch + b0 + kb2, G)]
                pltpu.make_async_copy(
                    rows_v.at[t2 % NBUF], dst2, out_sems.at[t2 % NBUF]
                ).wait()

            issue_gather(t + 2)

        return carry

    issue_gather(0)
    issue_gather(1)
    lax.fori_loop(0, nsteps, step, 0)

    for tl in range(nsteps - NBUF, nsteps):
        h2 = tl // kph
        kb2 = (tl % kph) * G
        src = out_hbm.at[pl.ds(h2 * batch + b0 + kb2, G)]
        pltpu.make_async_copy(
            rows_v.at[tl % NBUF], src, out_sems.at[tl % NBUF]
        ).wait()


@functools.partial(jax.jit, static_argnames=("bpw",))
def _embed_call(xt, w, bpw):
    batch = bpw * NW
    mesh = plsc.VectorSubcoreMesh(core_axis_name="c", subcore_axis_name="s")
    return pl.kernel(
        functools.partial(_body, bpw=bpw),
        out_type=jax.ShapeDtypeStruct((HIST * batch, EMBED), jnp.float32),
        mesh=mesh,
        scratch_types=[
            pltpu.VMEM((HIST, bpw), jnp.int32),
            pltpu.VMEM((NBUF, G, EMBED), jnp.float32),
            pltpu.SemaphoreType.DMA((NBUF,)),
            pltpu.SemaphoreType.DMA((NBUF,)),
        ],
        compiler_params=pltpu.CompilerParams(use_tc_tiling_on_sc=False),
    )(xt, w)


def kernel(x, W):
    b, h, _ = x.shape
    assert h == HIST and b % (NW * G) == 0
    bpw = b // NW
    xt = jnp.transpose(x, (1, 0, 2))[:, :, 0].astype(jnp.int32)
    w2 = jnp.pad(W, ((0, 0), (0, EMBED))).reshape(2 * W.shape[0], EMBED)
    out = _embed_call(xt, w2, bpw)
    return jnp.transpose(out.reshape(HIST, b, EMBED), (1, 0, 2))
